# baseline (speedup 1.0000x reference)
import functools

import jax
import jax.numpy as jnp
from jax import lax
from jax.experimental import pallas as pl
from jax.experimental.pallas import tpu as pltpu
from jax.experimental.pallas import tpu_sc as plsc

N = 10000
D = 128
G = 64
L = 3

NC = 2
NS = 16
NW = NC * NS
K = 128
EPT = 10240
EPAD = NW * EPT
CHUNKS = EPT // K
ACC_ROWS = 10112
ZROWS = ACC_ROWS // NS


NBUF = 2
TCH = EPAD // K
CPT = TCH // NW
CH_R = 40


def _segsum_body(h_hbm, src_hbm, dst_hbm, zeros_hbm,
                 out0_hbm, out1_hbm,
                 acc, src2d, dst2d, rows0, rows1, gsem, ssem):
    rows = [rows0, rows1]
    c = lax.axis_index("c")
    s = lax.axis_index("s")
    wid = s * NC + c

    pltpu.sync_copy(zeros_hbm, acc.at[pl.ds(s * ZROWS, ZROWS)])
    plsc.subcore_barrier()

    base = wid * CPT

    def run_edges():
        def refill(r, carry):
            off = base + r * CH_R
            pltpu.sync_copy(src_hbm.at[pl.ds(off, CH_R)], src2d)
            pltpu.sync_copy(dst_hbm.at[pl.ds(off, CH_R)], dst2d)
            for b in range(NBUF):
                pltpu.async_copy(h_hbm.at[src2d.at[b]], rows[b], gsem)

            def body(j, carry2):
                i0 = j * NBUF
                for b in range(NBUF):
                    i = i0 + b
                    pltpu.make_async_copy(h_hbm.at[src2d.at[i]], rows[b],
                                          gsem).wait()
                    pltpu.async_copy(rows[b], acc.at[dst2d.at[i]], ssem,
                                     add=True)
                for b in range(NBUF):
                    i = i0 + b
                    pltpu.make_async_copy(rows[b], acc.at[dst2d.at[i]],
                                          ssem).wait()

                    @pl.when(i + NBUF < CH_R)
                    def _():
                        pltpu.async_copy(h_hbm.at[src2d.at[i + NBUF]],
                                         rows[b], gsem)

                return carry2

            lax.fori_loop(0, CH_R // NBUF, body, 0)
            return carry

        lax.fori_loop(0, CPT // CH_R, refill, 0)

    run_edges()
    plsc.subcore_barrier()

    @pl.when(c == 0)
    def _():
        pltpu.sync_copy(acc.at[pl.ds(s * ZROWS, ZROWS)],
                        out0_hbm.at[pl.ds(s * ZROWS, ZROWS)])

    @pl.when(c == 1)
    def _():
        pltpu.sync_copy(acc.at[pl.ds(s * ZROWS, ZROWS)],
                        out1_hbm.at[pl.ds(s * ZROWS, ZROWS)])


def _make_segsum():
    mesh = plsc.VectorSubcoreMesh(core_axis_name="c", subcore_axis_name="s")
    return pl.kernel(
        _segsum_body,
        out_type=[jax.ShapeDtypeStruct((ACC_ROWS, D), jnp.float32),
                  jax.ShapeDtypeStruct((ACC_ROWS, D), jnp.float32)],
        mesh=mesh,
        scratch_types=[
            pltpu.VMEM_SHARED((ACC_ROWS, D), jnp.float32),
            pltpu.VMEM((CH_R, K), jnp.int32),
            pltpu.VMEM((CH_R, K), jnp.int32),
            pltpu.VMEM((K, D), jnp.float32),
            pltpu.VMEM((K, D), jnp.float32),
            pltpu.SemaphoreType.DMA,
            pltpu.SemaphoreType.DMA,
        ],
    )


NB = 10
BR = N // NB


def _mlp_body(h_ref, a0_ref, a1_ref, w1_ref, b1_ref, w2_ref, b2_ref,
              z_ref, sum_ref, sq_ref):
    @pl.when(pl.program_id(0) == 0)
    def _():
        sum_ref[...] = jnp.zeros_like(sum_ref)
        sq_ref[...] = jnp.zeros_like(sq_ref)

    z = h_ref[...] + a0_ref[...] + a1_ref[...]
    z1 = jnp.maximum(
        jnp.dot(z, w1_ref[...], preferred_element_type=jnp.float32)
        + b1_ref[...], 0.0)
    z2 = (jnp.dot(z1, w2_ref[...], preferred_element_type=jnp.float32)
          + b2_ref[...])
    z_ref[...] = z2
    z3 = z2.reshape(BR // 8, 8, D)
    sum_ref[...] += jnp.sum(z3, axis=0)
    sq_ref[...] += jnp.sum(z3 * z3, axis=0)


def _mlp(h, a0, a1, w1, b1, w2, b2):
    row = pl.BlockSpec((BR, D), lambda i: (i, 0))
    full = pl.BlockSpec((D, D), lambda i: (0, 0))
    vec = pl.BlockSpec((1, D), lambda i: (0, 0))
    stat = pl.BlockSpec((8, D), lambda i: (0, 0))
    return pl.pallas_call(
        _mlp_body,
        grid=(NB,),
        in_specs=[row, row, row, full, vec, full, vec],
        out_specs=[row, stat, stat],
        out_shape=[jax.ShapeDtypeStruct((N, D), jnp.float32),
                   jax.ShapeDtypeStruct((8, D), jnp.float32),
                   jax.ShapeDtypeStruct((8, D), jnp.float32)],
    )(h, a0, a1, w1, b1.reshape(1, D), w2, b2.reshape(1, D))


def _bn_body(z_ref, sum_ref, sq_ref, gamma_ref, beta_ref, out_ref):
    ssum = jnp.sum(sum_ref[...], axis=0, keepdims=True)
    ssq = jnp.sum(sq_ref[...], axis=0, keepdims=True)
    mu = ssum / N
    var = ssq / N - mu * mu
    inv = gamma_ref[...] * lax.rsqrt(var + 1e-5)
    out_ref[...] = jnp.maximum((z_ref[...] - mu) * inv + beta_ref[...], 0.0)


def _bn(z, ssum, ssq, gamma, beta):
    row = pl.BlockSpec((BR, D), lambda i: (i, 0))
    stat = pl.BlockSpec((8, D), lambda i: (0, 0))
    vec = pl.BlockSpec((1, D), lambda i: (0, 0))
    return pl.pallas_call(
        _bn_body,
        grid=(NB,),
        in_specs=[row, stat, stat, vec, vec],
        out_specs=row,
        out_shape=jax.ShapeDtypeStruct((N, D), jnp.float32),
    )(z, ssum, ssq, gamma.reshape(1, D), beta.reshape(1, D))


def _pool_body(h_ref, batch_ref, wp_ref, bp_ref, out_ref):
    gids = lax.broadcasted_iota(jnp.int32, (G, N), 0)
    onehot = (batch_ref[...] == gids).astype(jnp.float32)
    sums = jnp.dot(onehot, h_ref[...], preferred_element_type=jnp.float32,
                   precision=lax.Precision.HIGHEST)
    cnts = jnp.sum(onehot, axis=1, keepdims=True)
    hg = sums / jnp.maximum(cnts, 1.0)
    out_ref[...] = (jnp.dot(hg, wp_ref[...], preferred_element_type=jnp.float32)
                    + bp_ref[...])


def _pool(h, batch, wp, bp):
    return pl.pallas_call(
        _pool_body,
        out_shape=jax.ShapeDtypeStruct((G, D), jnp.float32),
    )(h, batch.reshape(1, N), wp, bp.reshape(1, D))


def kernel(x, edge_index, batch,
           W1_0, b1_0, W2_0, b2_0, gamma_0, beta_0,
           W1_1, b1_1, W2_1, b2_1, gamma_1, beta_1,
           W1_2, b1_2, W2_2, b2_2, gamma_2, beta_2,
           Wp, bp):
    E = edge_index.shape[1]
    pad = EPAD - E
    src = jnp.concatenate([edge_index[0], jnp.zeros((pad,), jnp.int32)])
    dst = jnp.concatenate([edge_index[1], jnp.full((pad,), N, jnp.int32)])
    src = src.reshape(TCH, K)
    dst = dst.reshape(TCH, K)
    zeros = jnp.zeros((ZROWS, D), jnp.float32)

    segsum = _make_segsum()
    params = [(W1_0, b1_0, W2_0, b2_0, gamma_0, beta_0),
              (W1_1, b1_1, W2_1, b2_1, gamma_1, beta_1),
              (W1_2, b1_2, W2_2, b2_2, gamma_2, beta_2)]

    h = x
    for l in range(L):
        w1, b1, w2, b2, gamma, beta = params[l]
        a0, a1 = segsum(h, src, dst, zeros)
        z, ssum, ssq = _mlp(h, a0[:N], a1[:N], w1, b1, w2, b2)
        h = _bn(z, ssum, ssq, gamma, beta)
    return _pool(h, batch, Wp, bp)


# speedup vs baseline: 1.2258x; 1.2258x over previous
import functools

import jax
import jax.numpy as jnp
from jax import lax
from jax.experimental import pallas as pl
from jax.experimental.pallas import tpu as pltpu
from jax.experimental.pallas import tpu_sc as plsc

N = 10000
D = 128
G = 64
L = 3

NC = 2
NS = 16
NW = NC * NS
K = 128
EPT = 10240
EPAD = NW * EPT
CHUNKS = EPT // K
ACC_ROWS = 10112
ZROWS = ACC_ROWS // NS


NBUF = 2
TCH = EPAD // K
CPT = TCH // NW
CH_R = 40


def _segsum_body(h_hbm, src_hbm, dst_hbm, zeros_hbm,
                 out0_hbm, out1_hbm,
                 acc, src2d, dst2d, rows0, rows1, gsem, ssem):
    rows = [rows0, rows1]
    c = lax.axis_index("c")
    s = lax.axis_index("s")
    wid = s * NC + c

    pltpu.sync_copy(zeros_hbm, acc.at[pl.ds(s * ZROWS, ZROWS)])
    plsc.subcore_barrier()

    def run_edges():
        for half in range(CPT // CH_R):
            pltpu.sync_copy(src_hbm.at[wid, half], src2d)
            pltpu.sync_copy(dst_hbm.at[wid, half], dst2d)
            for b in range(NBUF):
                pltpu.async_copy(h_hbm.at[src2d.at[b]], rows[b], gsem)

            def body(j, carry2):
                i0 = j * NBUF
                for b in range(NBUF):
                    i = i0 + b
                    pltpu.make_async_copy(h_hbm.at[src2d.at[i]], rows[b],
                                          gsem).wait()
                    pltpu.async_copy(rows[b], acc.at[dst2d.at[i]], ssem,
                                     add=True)
                for b in range(NBUF):
                    i = i0 + b
                    pltpu.make_async_copy(rows[b], acc.at[dst2d.at[i]],
                                          ssem).wait()

                    @pl.when(i + NBUF < CH_R)
                    def _():
                        pltpu.async_copy(h_hbm.at[src2d.at[i + NBUF]],
                                         rows[b], gsem)

                return carry2

            lax.fori_loop(0, CH_R // NBUF, body, 0)

    run_edges()
    plsc.subcore_barrier()

    @pl.when(c == 0)
    def _():
        pltpu.sync_copy(acc.at[pl.ds(s * ZROWS, ZROWS)],
                        out0_hbm.at[pl.ds(s * ZROWS, ZROWS)])

    @pl.when(c == 1)
    def _():
        pltpu.sync_copy(acc.at[pl.ds(s * ZROWS, ZROWS)],
                        out1_hbm.at[pl.ds(s * ZROWS, ZROWS)])


def _make_segsum():
    mesh = plsc.VectorSubcoreMesh(core_axis_name="c", subcore_axis_name="s")
    return pl.kernel(
        _segsum_body,
        out_type=[jax.ShapeDtypeStruct((ACC_ROWS, D), jnp.float32),
                  jax.ShapeDtypeStruct((ACC_ROWS, D), jnp.float32)],
        mesh=mesh,
        scratch_types=[
            pltpu.VMEM_SHARED((ACC_ROWS, D), jnp.float32),
            pltpu.VMEM((CH_R, K), jnp.int32),
            pltpu.VMEM((CH_R, K), jnp.int32),
            pltpu.VMEM((K, D), jnp.float32),
            pltpu.VMEM((K, D), jnp.float32),
            pltpu.SemaphoreType.DMA,
            pltpu.SemaphoreType.DMA,
        ],
    )


NB = 10
BR = N // NB


def _mlp_body(h_ref, a0_ref, a1_ref, w1_ref, b1_ref, w2_ref, b2_ref,
              z_ref, sum_ref, sq_ref):
    @pl.when(pl.program_id(0) == 0)
    def _():
        sum_ref[...] = jnp.zeros_like(sum_ref)
        sq_ref[...] = jnp.zeros_like(sq_ref)

    z = h_ref[...] + a0_ref[...] + a1_ref[...]
    z1 = jnp.maximum(
        jnp.dot(z, w1_ref[...], preferred_element_type=jnp.float32)
        + b1_ref[...], 0.0)
    z2 = (jnp.dot(z1, w2_ref[...], preferred_element_type=jnp.float32)
          + b2_ref[...])
    z_ref[...] = z2
    z3 = z2.reshape(BR // 8, 8, D)
    sum_ref[...] += jnp.sum(z3, axis=0)
    sq_ref[...] += jnp.sum(z3 * z3, axis=0)


def _mlp(h, a0, a1, w1, b1, w2, b2):
    row = pl.BlockSpec((BR, D), lambda i: (i, 0))
    full = pl.BlockSpec((D, D), lambda i: (0, 0))
    vec = pl.BlockSpec((1, D), lambda i: (0, 0))
    stat = pl.BlockSpec((8, D), lambda i: (0, 0))
    return pl.pallas_call(
        _mlp_body,
        grid=(NB,),
        in_specs=[row, row, row, full, vec, full, vec],
        out_specs=[row, stat, stat],
        out_shape=[jax.ShapeDtypeStruct((N, D), jnp.float32),
                   jax.ShapeDtypeStruct((8, D), jnp.float32),
                   jax.ShapeDtypeStruct((8, D), jnp.float32)],
    )(h, a0, a1, w1, b1.reshape(1, D), w2, b2.reshape(1, D))


def _bn_body(z_ref, sum_ref, sq_ref, gamma_ref, beta_ref, out_ref):
    ssum = jnp.sum(sum_ref[...], axis=0, keepdims=True)
    ssq = jnp.sum(sq_ref[...], axis=0, keepdims=True)
    mu = ssum / N
    var = ssq / N - mu * mu
    inv = gamma_ref[...] * lax.rsqrt(var + 1e-5)
    out_ref[...] = jnp.maximum((z_ref[...] - mu) * inv + beta_ref[...], 0.0)


def _bn(z, ssum, ssq, gamma, beta):
    row = pl.BlockSpec((BR, D), lambda i: (i, 0))
    stat = pl.BlockSpec((8, D), lambda i: (0, 0))
    vec = pl.BlockSpec((1, D), lambda i: (0, 0))
    return pl.pallas_call(
        _bn_body,
        grid=(NB,),
        in_specs=[row, stat, stat, vec, vec],
        out_specs=row,
        out_shape=jax.ShapeDtypeStruct((N, D), jnp.float32),
    )(z, ssum, ssq, gamma.reshape(1, D), beta.reshape(1, D))


def _pool_body(h_ref, batch_ref, wp_ref, bp_ref, out_ref):
    gids = lax.broadcasted_iota(jnp.int32, (G, N), 0)
    onehot = (batch_ref[...] == gids).astype(jnp.float32)
    sums = jnp.dot(onehot, h_ref[...], preferred_element_type=jnp.float32,
                   precision=lax.Precision.HIGHEST)
    cnts = jnp.sum(onehot, axis=1, keepdims=True)
    hg = sums / jnp.maximum(cnts, 1.0)
    out_ref[...] = (jnp.dot(hg, wp_ref[...], preferred_element_type=jnp.float32)
                    + bp_ref[...])


def _pool(h, batch, wp, bp):
    return pl.pallas_call(
        _pool_body,
        out_shape=jax.ShapeDtypeStruct((G, D), jnp.float32),
    )(h, batch.reshape(1, N), wp, bp.reshape(1, D))


def kernel(x, edge_index, batch,
           W1_0, b1_0, W2_0, b2_0, gamma_0, beta_0,
           W1_1, b1_1, W2_1, b2_1, gamma_1, beta_1,
           W1_2, b1_2, W2_2, b2_2, gamma_2, beta_2,
           Wp, bp):
    E = edge_index.shape[1]
    pad = EPAD - E
    src = jnp.concatenate([edge_index[0], jnp.zeros((pad,), jnp.int32)])
    dst = jnp.concatenate([edge_index[1], jnp.full((pad,), N, jnp.int32)])
    src = src.reshape(NW, CPT // CH_R, CH_R, K)
    dst = dst.reshape(NW, CPT // CH_R, CH_R, K)
    zeros = jnp.zeros((ZROWS, D), jnp.float32)

    segsum = _make_segsum()
    params = [(W1_0, b1_0, W2_0, b2_0, gamma_0, beta_0),
              (W1_1, b1_1, W2_1, b2_1, gamma_1, beta_1),
              (W1_2, b1_2, W2_2, b2_2, gamma_2, beta_2)]

    h = x
    for l in range(L):
        w1, b1, w2, b2, gamma, beta = params[l]
        a0, a1 = segsum(h, src, dst, zeros)
        z, ssum, ssq = _mlp(h, a0[:N], a1[:N], w1, b1, w2, b2)
        h = _bn(z, ssum, ssq, gamma, beta)
    return _pool(h, batch, Wp, bp)
